# baseline (speedup 1.0000x reference)
import functools

import jax
import jax.numpy as jnp
from jax import lax
from jax.experimental import pallas as pl
from jax.experimental.pallas import tpu as pltpu
from jax.experimental.pallas import tpu_sc as plsc

N = 10000
E = 320000
D = 128
H = 128

NC = 2
NS = 16
NW = NC * NS
EP = E // NW
K = 80
NCH = EP // K
NP = 10112
SL = NP // NS
DW = 16

_mesh = plsc.VectorSubcoreMesh(core_axis_name="c", subcore_axis_name="s")


@functools.partial(
    pl.kernel,
    out_type=jax.ShapeDtypeStruct((NW, NP), jnp.float32),
    mesh=_mesh,
    compiler_params=pltpu.CompilerParams(needs_layout_passes=False),
    scratch_types=[
        pltpu.VMEM((EP,), jnp.int32),
        pltpu.VMEM((NP,), jnp.float32),
    ],
)
def _deg_kernel(dst_hbm, out_hbm, dst_v, deg_v):
    c = lax.axis_index("c")
    s = lax.axis_index("s")
    wid = c * NS + s
    pltpu.sync_copy(dst_hbm.at[wid], dst_v)

    def zero_body(i, carry):
        deg_v[pl.ds(i * 16, 16)] = jnp.zeros((16,), jnp.float32)
        return carry

    lax.fori_loop(0, NP // 16, zero_body, 0)

    def body(i, carry):
        idx = dst_v[pl.ds(i * 16, 16)]
        plsc.addupdate_scatter(deg_v, [idx], jnp.ones((16,), jnp.float32))
        return carry

    lax.fori_loop(0, EP // 16, body, 0)
    pltpu.sync_copy(deg_v, out_hbm.at[wid])


@functools.partial(
    pl.kernel,
    out_type=jax.ShapeDtypeStruct((NC, NP, H), jnp.float32),
    mesh=_mesh,
    scratch_types=[
        pltpu.VMEM((EP,), jnp.int32),
        pltpu.VMEM((NCH, K), jnp.int32),
        pltpu.VMEM((K, H), jnp.float32),
        pltpu.VMEM((K, H), jnp.float32),
        pltpu.VMEM_SHARED((NP, H), jnp.float32),
        pltpu.SemaphoreType.DMA,
        pltpu.SemaphoreType.DMA,
    ],
)
def _accum_kernel(h_hbm, src_hbm, dst_hbm, z_hbm, out_hbm,
                  src_v, dst_v, rows0_v, rows1_v, acc_sh, gsem0, gsem1):
    c = lax.axis_index("c")
    s = lax.axis_index("s")
    wid = c * NS + s
    pltpu.sync_copy(src_hbm.at[wid], src_v)
    pltpu.sync_copy(dst_hbm.at[wid], dst_v)
    pltpu.sync_copy(z_hbm, acc_sh.at[pl.ds(s * SL, SL)])
    plsc.subcore_barrier()

    rows = (rows0_v, rows1_v)
    gsem = (gsem0, gsem1)

    def gidx(j):
        return src_v.at[pl.ds(j * K, K)]

    pltpu.async_copy(h_hbm.at[gidx(0)], rows[0], gsem[0])
    pltpu.async_copy(h_hbm.at[gidx(1)], rows[1], gsem[1])

    def step(j, b):
        pltpu.make_async_copy(h_hbm.at[gidx(j)], rows[b], gsem[b]).wait()
        pltpu.sync_copy(rows[b], acc_sh.at[dst_v.at[j]], add=True)

        @pl.when(j < NCH - 2)
        def _():
            pltpu.async_copy(h_hbm.at[gidx(j + 2)], rows[b], gsem[b])

    def body(i, carry):
        for b in range(2):
            step(2 * i + b, b)
        return carry

    lax.fori_loop(0, NCH // 2, body, 0)
    if NCH % 2:
        step(NCH - 1, (NCH - 1) % 2)
    plsc.subcore_barrier()
    pltpu.sync_copy(acc_sh.at[pl.ds(s * SL, SL)],
                    out_hbm.at[c, pl.ds(s * SL, SL)])


RB = 1000
GRID = N // RB


def _tc1_body(x_ref, w_ref, degw_ref, h_ref, dis_ref):
    deg = jnp.sum(degw_ref[..., 0], axis=0)
    dis = jnp.where(deg > 0.0, lax.rsqrt(deg), 0.0)
    h = jnp.dot(x_ref[...], w_ref[...], preferred_element_type=jnp.float32)
    h_ref[...] = dis[:, None] * h
    dis_ref[...] = dis[:, None]


_tc1 = pl.pallas_call(
    _tc1_body,
    grid=(GRID,),
    in_specs=[
        pl.BlockSpec((RB, D), lambda i: (i, 0)),
        pl.BlockSpec((D, H), lambda i: (0, 0)),
        pl.BlockSpec((NW, RB, 1), lambda i: (0, i, 0)),
    ],
    out_specs=[
        pl.BlockSpec((RB, H), lambda i: (i, 0)),
        pl.BlockSpec((RB, 1), lambda i: (i, 0)),
    ],
    out_shape=[
        jax.ShapeDtypeStruct((N, H), jnp.float32),
        jax.ShapeDtypeStruct((N, 1), jnp.float32),
    ],
)


def _tc2_body(acc_ref, dis_ref, b1_ref, w2_ref, out_ref):
    a = acc_ref[0] + acc_ref[1]
    dis = dis_ref[...]
    h2 = jax.nn.relu(dis * a + b1_ref[...])
    out_ref[...] = dis * jnp.dot(h2, w2_ref[...],
                                 preferred_element_type=jnp.float32)


_tc2 = pl.pallas_call(
    _tc2_body,
    grid=(GRID,),
    in_specs=[
        pl.BlockSpec((NC, RB, H), lambda i: (0, i, 0)),
        pl.BlockSpec((RB, 1), lambda i: (i, 0)),
        pl.BlockSpec((1, H), lambda i: (0, 0)),
        pl.BlockSpec((H, H), lambda i: (0, 0)),
    ],
    out_specs=pl.BlockSpec((RB, H), lambda i: (i, 0)),
    out_shape=jax.ShapeDtypeStruct((N, H), jnp.float32),
)


def _tc3_body(acc_ref, dis_ref, b2_ref, out_ref):
    a = acc_ref[0] + acc_ref[1]
    out_ref[...] = dis_ref[...] * a + b2_ref[...]


_tc3 = pl.pallas_call(
    _tc3_body,
    grid=(GRID,),
    in_specs=[
        pl.BlockSpec((NC, RB, H), lambda i: (0, i, 0)),
        pl.BlockSpec((RB, 1), lambda i: (i, 0)),
        pl.BlockSpec((1, H), lambda i: (0, 0)),
    ],
    out_specs=pl.BlockSpec((RB, H), lambda i: (i, 0)),
    out_shape=jax.ShapeDtypeStruct((N, H), jnp.float32),
)


def kernel(x, edge_index, W1, b1, W2, b2):
    src = edge_index[0].reshape(NW, EP)
    dst = edge_index[1].reshape(NW, NCH, K)
    dst_flat = edge_index[1].reshape(NW, EP)
    zrow = jnp.zeros((SL, H), jnp.float32)

    degw = _deg_kernel(dst_flat).reshape(NW, NP, 1)
    h1p, dis = _tc1(x, W1, degw)
    a1 = _accum_kernel(h1p, src, dst, zrow)
    h2p = _tc2(a1, dis, b1.reshape(1, H), W2)
    a2 = _accum_kernel(h2p, src, dst, zrow)
    return _tc3(a2, dis, b2.reshape(1, H))


# speedup vs baseline: 1.3726x; 1.3726x over previous
import functools

import jax
import jax.numpy as jnp
from jax import lax
from jax.experimental import pallas as pl
from jax.experimental.pallas import tpu as pltpu
from jax.experimental.pallas import tpu_sc as plsc

N = 10000
E = 320000
D = 128
H = 128

NC = 2
NS = 16
NW = NC * NS
EP = E // NW
K = 80
NCH = EP // K
NP = 10112
SL = NP // NS
DW = 16

_mesh = plsc.VectorSubcoreMesh(core_axis_name="c", subcore_axis_name="s")


@functools.partial(
    pl.kernel,
    out_type=jax.ShapeDtypeStruct((NW, NP), jnp.float32),
    mesh=_mesh,
    compiler_params=pltpu.CompilerParams(needs_layout_passes=False),
    scratch_types=[
        pltpu.VMEM((EP,), jnp.int32),
        pltpu.VMEM((NP,), jnp.float32),
    ],
)
def _deg_kernel(dst_hbm, out_hbm, dst_v, deg_v):
    c = lax.axis_index("c")
    s = lax.axis_index("s")
    wid = c * NS + s
    pltpu.sync_copy(dst_hbm.at[wid], dst_v)

    def zero_body(i, carry):
        deg_v[pl.ds(i * 16, 16)] = jnp.zeros((16,), jnp.float32)
        return carry

    lax.fori_loop(0, NP // 16, zero_body, 0)

    def body(i, carry):
        idx = dst_v[pl.ds(i * 16, 16)]
        plsc.addupdate_scatter(deg_v, [idx], jnp.ones((16,), jnp.float32))
        return carry

    lax.fori_loop(0, EP // 16, body, 0)
    pltpu.sync_copy(deg_v, out_hbm.at[wid])


@functools.partial(
    pl.kernel,
    out_type=jax.ShapeDtypeStruct((NC, NP, H), jnp.float32),
    mesh=_mesh,
    scratch_types=[
        pltpu.VMEM((EP,), jnp.int32),
        pltpu.VMEM((NCH, K), jnp.int32),
        pltpu.VMEM((K, H), jnp.float32),
        pltpu.VMEM((K, H), jnp.float32),
        pltpu.VMEM_SHARED((NP, H), jnp.float32),
        pltpu.SemaphoreType.DMA,
        pltpu.SemaphoreType.DMA,
    ],
)
def _accum_kernel(h_hbm, src_hbm, dst_hbm, z_hbm, out_hbm,
                  src_v, dst_v, rows0_v, rows1_v, acc_sh, gsem0, gsem1):
    c = lax.axis_index("c")
    s = lax.axis_index("s")
    wid = c * NS + s
    pltpu.sync_copy(src_hbm.at[wid], src_v)
    pltpu.sync_copy(dst_hbm.at[wid], dst_v)
    pltpu.sync_copy(z_hbm, acc_sh.at[pl.ds(s * SL, SL)])
    plsc.subcore_barrier()

    rows = (rows0_v, rows1_v)
    gsem = (gsem0, gsem1)

    def gidx(j):
        return src_v.at[pl.ds(j * K, K)]

    pltpu.async_copy(h_hbm.at[gidx(0)], rows[0], gsem[0])
    pltpu.async_copy(h_hbm.at[gidx(1)], rows[1], gsem[1])

    def step(j, b):
        pltpu.make_async_copy(h_hbm.at[gidx(j)], rows[b], gsem[b]).wait()
        pltpu.sync_copy(rows[b], acc_sh.at[dst_v.at[j]], add=True)

        @pl.when(j < NCH - 2)
        def _():
            pltpu.async_copy(h_hbm.at[gidx(j + 2)], rows[b], gsem[b])

    def body(i, carry):
        for b in range(2):
            step(2 * i + b, b)
        return carry

    lax.fori_loop(0, NCH // 2, body, 0)
    if NCH % 2:
        step(NCH - 1, (NCH - 1) % 2)
    plsc.subcore_barrier()
    pltpu.sync_copy(acc_sh.at[pl.ds(s * SL, SL)],
                    out_hbm.at[c, pl.ds(s * SL, SL)])


RB = 1000
GRID = N // RB


def _tc1_body(x_ref, w_ref, degw_ref, h_ref, dis_ref):
    deg = jnp.sum(degw_ref[...], axis=1)
    dis = jnp.where(deg > 0.0, lax.rsqrt(deg), 0.0)
    h = jnp.dot(x_ref[...], w_ref[...], preferred_element_type=jnp.float32)
    h_ref[...] = dis[:, None] * h
    dis_ref[...] = dis[:, None]


_tc1 = pl.pallas_call(
    _tc1_body,
    grid=(GRID,),
    in_specs=[
        pl.BlockSpec((RB, D), lambda i: (i, 0)),
        pl.BlockSpec((D, H), lambda i: (0, 0)),
        pl.BlockSpec((RB, NW), lambda i: (i, 0)),
    ],
    out_specs=[
        pl.BlockSpec((RB, H), lambda i: (i, 0)),
        pl.BlockSpec((RB, 1), lambda i: (i, 0)),
    ],
    out_shape=[
        jax.ShapeDtypeStruct((N, H), jnp.float32),
        jax.ShapeDtypeStruct((N, 1), jnp.float32),
    ],
)


def _tc2_body(acc_ref, dis_ref, b1_ref, w2_ref, out_ref):
    a = acc_ref[0] + acc_ref[1]
    dis = dis_ref[...]
    h2 = jax.nn.relu(dis * a + b1_ref[...])
    out_ref[...] = dis * jnp.dot(h2, w2_ref[...],
                                 preferred_element_type=jnp.float32)


_tc2 = pl.pallas_call(
    _tc2_body,
    grid=(GRID,),
    in_specs=[
        pl.BlockSpec((NC, RB, H), lambda i: (0, i, 0)),
        pl.BlockSpec((RB, 1), lambda i: (i, 0)),
        pl.BlockSpec((1, H), lambda i: (0, 0)),
        pl.BlockSpec((H, H), lambda i: (0, 0)),
    ],
    out_specs=pl.BlockSpec((RB, H), lambda i: (i, 0)),
    out_shape=jax.ShapeDtypeStruct((N, H), jnp.float32),
)


def _tc3_body(acc_ref, dis_ref, b2_ref, out_ref):
    a = acc_ref[0] + acc_ref[1]
    out_ref[...] = dis_ref[...] * a + b2_ref[...]


_tc3 = pl.pallas_call(
    _tc3_body,
    grid=(GRID,),
    in_specs=[
        pl.BlockSpec((NC, RB, H), lambda i: (0, i, 0)),
        pl.BlockSpec((RB, 1), lambda i: (i, 0)),
        pl.BlockSpec((1, H), lambda i: (0, 0)),
    ],
    out_specs=pl.BlockSpec((RB, H), lambda i: (i, 0)),
    out_shape=jax.ShapeDtypeStruct((N, H), jnp.float32),
)


def kernel(x, edge_index, W1, b1, W2, b2):
    src = edge_index[0].reshape(NW, EP)
    dst = edge_index[1].reshape(NW, NCH, K)
    dst_flat = edge_index[1].reshape(NW, EP)
    zrow = jnp.zeros((SL, H), jnp.float32)

    degw = _deg_kernel(dst_flat).T
    h1p, dis = _tc1(x, W1, degw)
    a1 = _accum_kernel(h1p, src, dst, zrow)
    h2p = _tc2(a1, dis, b1.reshape(1, H), W2)
    a2 = _accum_kernel(h2p, src, dst, zrow)
    return _tc3(a2, dis, b2.reshape(1, H))
